# baseline (speedup 1.0000x reference)
import functools

import jax
import jax.numpy as jnp
from jax import lax
from jax.experimental import pallas as pl
from jax.experimental.pallas import tpu as pltpu
from jax.experimental.pallas import tpu_sc as plsc

BATCH = 1024
HALF = BATCH // 2
N_FACTORS = 64
HIDDEN = 128
N_PRODUCTS = 100000
BN = 4096
NBUF = 4


def _sc_gather(users, product_1, product_2, wu_t, wp_t, n_batch):
    info = plsc.get_sparse_core_info()
    nw = info.num_cores * info.num_subcores
    b_per_w = n_batch // nw
    n_grp = b_per_w // 16

    mesh = plsc.VectorSubcoreMesh(core_axis_name="c", subcore_axis_name="s")

    @functools.partial(
        pl.kernel,
        mesh=mesh,
        out_type=[jax.ShapeDtypeStruct((n_batch, N_FACTORS), jnp.float32)] * 3,
        scratch_types=[
            pltpu.VMEM((b_per_w,), jnp.int32),
            pltpu.VMEM((b_per_w, N_FACTORS), jnp.float32),
        ]
        + [pltpu.VMEM((N_FACTORS, 128), jnp.float32) for _ in range(NBUF)]
        + [pltpu.SemaphoreType.DMA for _ in range(NBUF)],
        compiler_params=pltpu.CompilerParams(needs_layout_passes=False),
    )
    def gather_kernel(users_h, p1_h, p2_h, wu_h, wp_h,
                      eu_o, e1_o, e2_o,
                      idx_v, rows_v, *bufs_sems):
        bufs = bufs_sems[:NBUF]
        sems = bufs_sems[NBUF:]
        wid = lax.axis_index("s") * info.num_cores + lax.axis_index("c")
        base = wid * b_per_w
        lanes = lax.iota(jnp.int32, 16)

        for idx_h, tab_h, out_h in ((users_h, wu_h, eu_o),
                                    (p1_h, wp_h, e1_o),
                                    (p2_h, wp_h, e2_o)):
            pltpu.sync_copy(idx_h.at[pl.ds(base, b_per_w)], idx_v)
            vecs = [idx_v[pl.ds(16 * g, 16)] for g in range(n_grp)]

            def block_of(i):
                vec = vecs[0]
                for g in range(1, n_grp):
                    vec = jnp.where(i < 16 * g, vec, vecs[g])
                lane = lax.rem(i, 16)
                return jnp.sum(jnp.where(lanes == lane, vec, 0))

            def issue(i, b):
                u = block_of(i)
                u_al = pl.multiple_of(lax.div(u, 128) * 128, 128)
                pltpu.async_copy(tab_h.at[:, pl.ds(u_al, 128)],
                                 bufs[b], sems[b])

            def extract(i, b):
                u = block_of(i)
                col = jnp.broadcast_to(lax.rem(u, 128), (16,)).astype(jnp.int32)
                pltpu.make_async_copy(tab_h.at[:, pl.ds(0, 128)],
                                      bufs[b], sems[b]).wait()
                for g in range(N_FACTORS // 16):
                    idx_d = lanes + g * 16
                    vals = plsc.load_gather(bufs[b], [idx_d, col])
                    rows_v[i, pl.ds(g * 16, 16)] = vals

            for b in range(NBUF):
                issue(jnp.int32(b), b)

            def body(k, carry):
                for b in range(NBUF):
                    extract(k * NBUF + b, b)
                for b in range(NBUF):
                    nxt = (k + 1) * NBUF + b
                    @pl.when(nxt < b_per_w)
                    def _issue_next(nxt=nxt, b=b):
                        issue(nxt, b)
                return carry

            lax.fori_loop(0, b_per_w // NBUF, body, 0)
            pltpu.sync_copy(rows_v, out_h.at[pl.ds(base, b_per_w)])

    return gather_kernel(users, product_1, product_2, wu_t, wp_t)


def _mlp_body(eu_ref, e1_ref, e2_ref, w1_ref, b1_ref, wfct_ref, bfc_ref,
              *rest):
    if len(rest) == 3:
        _, out_ref, ht_ref = rest
    else:
        out_ref, ht_ref = rest

    @pl.when(pl.program_id(0) == 0)
    def _():
        acc = jax.lax.dot_general(
            w1_ref[0:N_FACTORS, :], eu_ref[...],
            (((0,), (1,)), ((), ())), preferred_element_type=jnp.float32)
        acc += jax.lax.dot_general(
            w1_ref[N_FACTORS:2 * N_FACTORS, :], e1_ref[...],
            (((0,), (1,)), ((), ())), preferred_element_type=jnp.float32)
        acc += jax.lax.dot_general(
            w1_ref[2 * N_FACTORS:3 * N_FACTORS, :], e2_ref[...],
            (((0,), (1,)), ((), ())), preferred_element_type=jnp.float32)
        ht_ref[...] = jnp.maximum(acc + b1_ref[...], 0.0)

    bias_col = jax.lax.transpose(bfc_ref[...], (1, 0))
    out_ref[...] = jax.lax.dot_general(
        wfct_ref[...], ht_ref[...],
        (((1,), (0,)), ((), ())), preferred_element_type=jnp.float32,
    ) + bias_col


def _tc_mlp_half(eu, e1, e2, W1, b1c, WfcT, bfcc, half, prev=None):
    grid = (pl.cdiv(N_PRODUCTS, BN),)
    in_specs = [
        pl.BlockSpec((HALF, N_FACTORS), lambda j: (0, 0)),
        pl.BlockSpec((HALF, N_FACTORS), lambda j: (0, 0)),
        pl.BlockSpec((HALF, N_FACTORS), lambda j: (0, 0)),
        pl.BlockSpec((3 * N_FACTORS, HIDDEN), lambda j: (0, 0)),
        pl.BlockSpec((HIDDEN, 1), lambda j: (0, 0)),
        pl.BlockSpec((BN, HIDDEN), lambda j: (j, 0)),
        pl.BlockSpec((1, BN), lambda j: (0, j)),
    ]
    args = [eu, e1, e2, W1, b1c, WfcT, bfcc]
    aliases = {}
    if prev is not None:
        in_specs.append(pl.BlockSpec(memory_space=pl.ANY))
        args.append(prev)
        aliases = {7: 0}
    return pl.pallas_call(
        _mlp_body,
        grid=grid,
        in_specs=in_specs,
        out_specs=pl.BlockSpec((BN, HALF), lambda j, half=half: (j, half)),
        out_shape=jax.ShapeDtypeStruct((N_PRODUCTS, BATCH), jnp.float32),
        scratch_shapes=[pltpu.VMEM((HIDDEN, HALF), jnp.float32)],
        input_output_aliases=aliases,
        compiler_params=pltpu.CompilerParams(
            dimension_semantics=("arbitrary",),
        ),
    )(*args)


def kernel(users, product_1, product_2, Wu, Wp, W1, b1, Wfc, bfc):
    users = users.astype(jnp.int32)
    product_1 = product_1.astype(jnp.int32)
    product_2 = product_2.astype(jnp.int32)
    wu_t, wp_t = Wu.T, Wp.T
    wfct = Wfc.T
    b1c = b1.reshape(HIDDEN, 1)
    bfcr = bfc.reshape(1, N_PRODUCTS)

    halves = []
    for h in range(2):
        sl = slice(h * HALF, (h + 1) * HALF)
        halves.append(_sc_gather(users[sl], product_1[sl], product_2[sl],
                                 wu_t, wp_t, HALF))

    out = _tc_mlp_half(*halves[0], W1, b1c, wfct, bfcr, half=0)
    out = _tc_mlp_half(*halves[1], W1, b1c, wfct, bfcr, half=1, prev=out)
    return out.T


# speedup vs baseline: 1.1053x; 1.1053x over previous
import functools

import jax
import jax.numpy as jnp
from jax import lax
from jax.experimental import pallas as pl
from jax.experimental.pallas import tpu as pltpu
from jax.experimental.pallas import tpu_sc as plsc

BATCH = 1024
N_FACTORS = 64
HIDDEN = 128
N_PRODUCTS = 100000
BN = 4096
NBUF = 8


def _sc_gather(users, product_1, product_2, wu_t, wp_t):
    info = plsc.get_sparse_core_info()
    nw = info.num_cores * info.num_subcores
    b_per_w = BATCH // nw

    mesh = plsc.VectorSubcoreMesh(core_axis_name="c", subcore_axis_name="s")

    @functools.partial(
        pl.kernel,
        mesh=mesh,
        out_type=[jax.ShapeDtypeStruct((BATCH, N_FACTORS), jnp.float32)] * 3,
        scratch_types=[
            pltpu.VMEM((b_per_w,), jnp.int32),
            pltpu.VMEM((b_per_w, N_FACTORS), jnp.float32),
        ]
        + [pltpu.VMEM((N_FACTORS, 128), jnp.float32) for _ in range(NBUF)]
        + [pltpu.SemaphoreType.DMA for _ in range(NBUF)],
        compiler_params=pltpu.CompilerParams(needs_layout_passes=False),
    )
    def gather_kernel(users_h, p1_h, p2_h, wu_h, wp_h,
                      eu_o, e1_o, e2_o,
                      idx_v, rows_v, *bufs_sems):
        bufs = bufs_sems[:NBUF]
        sems = bufs_sems[NBUF:]
        wid = lax.axis_index("s") * info.num_cores + lax.axis_index("c")
        base = wid * b_per_w
        lanes = lax.iota(jnp.int32, 16)

        for idx_h, tab_h, out_h in ((users_h, wu_h, eu_o),
                                    (p1_h, wp_h, e1_o),
                                    (p2_h, wp_h, e2_o)):
            pltpu.sync_copy(idx_h.at[pl.ds(base, b_per_w)], idx_v)
            vec0 = idx_v[pl.ds(0, 16)]
            vec1 = idx_v[pl.ds(16, 16)]

            def block_of(i):
                vec = jnp.where(i < 16, vec0, vec1)
                lane = lax.rem(i, 16)
                u = jnp.sum(jnp.where(lanes == lane, vec, 0))
                return u

            def issue(i, b):
                u = block_of(i)
                u_al = pl.multiple_of(lax.div(u, 128) * 128, 128)
                pltpu.async_copy(tab_h.at[:, pl.ds(u_al, 128)],
                                 bufs[b], sems[b])

            def extract(i, b):
                u = block_of(i)
                col = jnp.broadcast_to(lax.rem(u, 128), (16,)).astype(jnp.int32)
                pltpu.make_async_copy(tab_h.at[:, pl.ds(0, 128)],
                                      bufs[b], sems[b]).wait()
                for g in range(N_FACTORS // 16):
                    idx_d = lanes + g * 16
                    vals = plsc.load_gather(bufs[b], [idx_d, col])
                    rows_v[i, pl.ds(g * 16, 16)] = vals

            for b in range(NBUF):
                issue(jnp.int32(b), b)

            def body(k, carry):
                for b in range(NBUF):
                    extract(k * NBUF + b, b)
                for b in range(NBUF):
                    nxt = (k + 1) * NBUF + b
                    @pl.when(nxt < b_per_w)
                    def _issue_next(nxt=nxt, b=b):
                        issue(nxt, b)
                return carry

            lax.fori_loop(0, b_per_w // NBUF, body, 0)
            pltpu.sync_copy(rows_v, out_h.at[pl.ds(base, b_per_w)])

    return gather_kernel(users, product_1, product_2, wu_t, wp_t)


def _mlp_body(eu_ref, e1_ref, e2_ref, w1_ref, b1_ref, wfct_ref, bfc_ref,
              out_ref, ht_ref):
    @pl.when(pl.program_id(0) == 0)
    def _():
        acc = jax.lax.dot_general(
            w1_ref[0:N_FACTORS, :], eu_ref[...],
            (((0,), (1,)), ((), ())), preferred_element_type=jnp.float32)
        acc += jax.lax.dot_general(
            w1_ref[N_FACTORS:2 * N_FACTORS, :], e1_ref[...],
            (((0,), (1,)), ((), ())), preferred_element_type=jnp.float32)
        acc += jax.lax.dot_general(
            w1_ref[2 * N_FACTORS:3 * N_FACTORS, :], e2_ref[...],
            (((0,), (1,)), ((), ())), preferred_element_type=jnp.float32)
        ht_ref[...] = jnp.maximum(acc + b1_ref[...], 0.0)

    bias_col = jax.lax.transpose(bfc_ref[...], (1, 0))
    out_ref[...] = jax.lax.dot_general(
        wfct_ref[...], ht_ref[...],
        (((1,), (0,)), ((), ())), preferred_element_type=jnp.float32,
    ) + bias_col


def _tc_mlp(eu, e1, e2, W1, b1c, WfcT, bfcc):
    grid = (pl.cdiv(N_PRODUCTS, BN),)
    return pl.pallas_call(
        _mlp_body,
        grid=grid,
        in_specs=[
            pl.BlockSpec((BATCH, N_FACTORS), lambda j: (0, 0)),
            pl.BlockSpec((BATCH, N_FACTORS), lambda j: (0, 0)),
            pl.BlockSpec((BATCH, N_FACTORS), lambda j: (0, 0)),
            pl.BlockSpec((3 * N_FACTORS, HIDDEN), lambda j: (0, 0)),
            pl.BlockSpec((HIDDEN, 1), lambda j: (0, 0)),
            pl.BlockSpec((BN, HIDDEN), lambda j: (j, 0)),
            pl.BlockSpec((1, BN), lambda j: (0, j)),
        ],
        out_specs=pl.BlockSpec((BN, BATCH), lambda j: (j, 0)),
        out_shape=jax.ShapeDtypeStruct((N_PRODUCTS, BATCH), jnp.float32),
        scratch_shapes=[pltpu.VMEM((HIDDEN, BATCH), jnp.float32)],
        compiler_params=pltpu.CompilerParams(
            dimension_semantics=("arbitrary",),
        ),
    )(eu, e1, e2, W1, b1c, WfcT, bfcc)


def kernel(users, product_1, product_2, Wu, Wp, W1, b1, Wfc, bfc):
    users = users.astype(jnp.int32)
    product_1 = product_1.astype(jnp.int32)
    product_2 = product_2.astype(jnp.int32)
    eu, e1, e2 = _sc_gather(users, product_1, product_2, Wu.T, Wp.T)
    out_t = _tc_mlp(eu, e1, e2, W1, b1.reshape(HIDDEN, 1),
                    Wfc.T, bfc.reshape(1, N_PRODUCTS))
    return out_t.T


# speedup vs baseline: 1.1195x; 1.0128x over previous
import functools

import jax
import jax.numpy as jnp
from jax import lax
from jax.experimental import pallas as pl
from jax.experimental.pallas import tpu as pltpu
from jax.experimental.pallas import tpu_sc as plsc

BATCH = 1024
N_FACTORS = 64
HIDDEN = 128
N_PRODUCTS = 100000
BN = 4096
NBUF = 12


def _sc_gather(users, product_1, product_2, wu_t, wp_t):
    info = plsc.get_sparse_core_info()
    nw = info.num_cores * info.num_subcores
    b_per_w = BATCH // nw

    mesh = plsc.VectorSubcoreMesh(core_axis_name="c", subcore_axis_name="s")

    @functools.partial(
        pl.kernel,
        mesh=mesh,
        out_type=[jax.ShapeDtypeStruct((BATCH, N_FACTORS), jnp.float32)] * 3,
        scratch_types=[
            pltpu.VMEM((b_per_w,), jnp.int32),
            pltpu.VMEM((b_per_w, N_FACTORS), jnp.float32),
        ]
        + [pltpu.VMEM((N_FACTORS, 128), jnp.float32) for _ in range(NBUF)]
        + [pltpu.SemaphoreType.DMA for _ in range(NBUF)],
        compiler_params=pltpu.CompilerParams(needs_layout_passes=False),
    )
    def gather_kernel(users_h, p1_h, p2_h, wu_h, wp_h,
                      eu_o, e1_o, e2_o,
                      idx_v, rows_v, *bufs_sems):
        bufs = bufs_sems[:NBUF]
        sems = bufs_sems[NBUF:]
        wid = lax.axis_index("s") * info.num_cores + lax.axis_index("c")
        base = wid * b_per_w
        lanes = lax.iota(jnp.int32, 16)

        for idx_h, tab_h, out_h in ((users_h, wu_h, eu_o),
                                    (p1_h, wp_h, e1_o),
                                    (p2_h, wp_h, e2_o)):
            pltpu.sync_copy(idx_h.at[pl.ds(base, b_per_w)], idx_v)
            vec0 = idx_v[pl.ds(0, 16)]
            vec1 = idx_v[pl.ds(16, 16)]

            def block_of(i):
                vec = jnp.where(i < 16, vec0, vec1)
                lane = lax.rem(i, 16)
                u = jnp.sum(jnp.where(lanes == lane, vec, 0))
                return u

            def issue(i, b):
                u = block_of(i)
                u_al = pl.multiple_of(lax.div(u, 128) * 128, 128)
                pltpu.async_copy(tab_h.at[:, pl.ds(u_al, 128)],
                                 bufs[b], sems[b])

            def extract(i, b):
                u = block_of(i)
                col = jnp.broadcast_to(lax.rem(u, 128), (16,)).astype(jnp.int32)
                pltpu.make_async_copy(tab_h.at[:, pl.ds(0, 128)],
                                      bufs[b], sems[b]).wait()
                for g in range(N_FACTORS // 16):
                    idx_d = lanes + g * 16
                    vals = plsc.load_gather(bufs[b], [idx_d, col])
                    rows_v[i, pl.ds(g * 16, 16)] = vals

            for b in range(NBUF):
                issue(jnp.int32(b), b)

            def body(k, carry):
                for b in range(NBUF):
                    extract(k * NBUF + b, b)
                for b in range(NBUF):
                    nxt = (k + 1) * NBUF + b
                    @pl.when(nxt < b_per_w)
                    def _issue_next(nxt=nxt, b=b):
                        issue(nxt, b)
                return carry

            lax.fori_loop(0, b_per_w // NBUF, body, 0)
            pltpu.sync_copy(rows_v, out_h.at[pl.ds(base, b_per_w)])

    return gather_kernel(users, product_1, product_2, wu_t, wp_t)


def _mlp_body(eu_ref, e1_ref, e2_ref, w1_ref, b1_ref, wfct_ref, bfc_ref,
              out_ref, ht_ref):
    @pl.when(pl.program_id(0) == 0)
    def _():
        acc = jax.lax.dot_general(
            w1_ref[0:N_FACTORS, :], eu_ref[...],
            (((0,), (1,)), ((), ())), preferred_element_type=jnp.float32)
        acc += jax.lax.dot_general(
            w1_ref[N_FACTORS:2 * N_FACTORS, :], e1_ref[...],
            (((0,), (1,)), ((), ())), preferred_element_type=jnp.float32)
        acc += jax.lax.dot_general(
            w1_ref[2 * N_FACTORS:3 * N_FACTORS, :], e2_ref[...],
            (((0,), (1,)), ((), ())), preferred_element_type=jnp.float32)
        ht_ref[...] = jnp.maximum(acc + b1_ref[...], 0.0)

    bias_col = jax.lax.transpose(bfc_ref[...], (1, 0))
    out_ref[...] = jax.lax.dot_general(
        wfct_ref[...], ht_ref[...],
        (((1,), (0,)), ((), ())), preferred_element_type=jnp.float32,
    ) + bias_col


def _tc_mlp(eu, e1, e2, W1, b1c, WfcT, bfcc):
    grid = (pl.cdiv(N_PRODUCTS, BN),)
    return pl.pallas_call(
        _mlp_body,
        grid=grid,
        in_specs=[
            pl.BlockSpec((BATCH, N_FACTORS), lambda j: (0, 0)),
            pl.BlockSpec((BATCH, N_FACTORS), lambda j: (0, 0)),
            pl.BlockSpec((BATCH, N_FACTORS), lambda j: (0, 0)),
            pl.BlockSpec((3 * N_FACTORS, HIDDEN), lambda j: (0, 0)),
            pl.BlockSpec((HIDDEN, 1), lambda j: (0, 0)),
            pl.BlockSpec((BN, HIDDEN), lambda j: (j, 0)),
            pl.BlockSpec((1, BN), lambda j: (0, j)),
        ],
        out_specs=pl.BlockSpec((BN, BATCH), lambda j: (j, 0)),
        out_shape=jax.ShapeDtypeStruct((N_PRODUCTS, BATCH), jnp.float32),
        scratch_shapes=[pltpu.VMEM((HIDDEN, BATCH), jnp.float32)],
        compiler_params=pltpu.CompilerParams(
            dimension_semantics=("arbitrary",),
        ),
    )(eu, e1, e2, W1, b1c, WfcT, bfcc)


def kernel(users, product_1, product_2, Wu, Wp, W1, b1, Wfc, bfc):
    users = users.astype(jnp.int32)
    product_1 = product_1.astype(jnp.int32)
    product_2 = product_2.astype(jnp.int32)
    eu, e1, e2 = _sc_gather(users, product_1, product_2, Wu.T, Wp.T)
    out_t = _tc_mlp(eu, e1, e2, W1, b1.reshape(HIDDEN, 1),
                    Wfc.T, bfc.reshape(1, N_PRODUCTS))
    return out_t.T
